# baseline (speedup 1.0000x reference)
import functools

import jax
import jax.numpy as jnp
from jax import lax
from jax.experimental import pallas as pl
from jax.experimental.pallas import tpu as pltpu
from jax.experimental.pallas import tpu_sc as plsc

_NC = 2
_NS = 16
_L = 16
_CHUNK = 128


@functools.cache
def _build(B, D):
    NW = _NC * _NS
    bpw = B // NW
    nch = bpw // _CHUNK
    groups = bpw // _L
    mesh = plsc.VectorSubcoreMesh(core_axis_name="c", subcore_axis_name="s",
                                  num_cores=_NC, num_subcores=_NS)

    @functools.partial(
        pl.kernel,
        out_type=jax.ShapeDtypeStruct((B,), jnp.float32),
        mesh=mesh,
        scratch_types=[
            pltpu.VMEM((nch, _CHUNK), jnp.int32),
            pltpu.VMEM((nch, _CHUNK), jnp.int32),
            pltpu.VMEM((bpw, D), jnp.float32),
            pltpu.VMEM((bpw, D), jnp.float32),
            pltpu.VMEM((bpw,), jnp.float32),
            pltpu.VMEM((bpw,), jnp.float32),
            pltpu.VMEM((bpw,), jnp.float32),
            pltpu.SemaphoreType.DMA,
        ],
        compiler_params=pltpu.CompilerParams(
            needs_layout_passes=False, use_tc_tiling_on_sc=False),
    )
    def sc_kernel(ut, it, ubt, ibt, uid, iid, out,
                  uidv, iidv, urow, irow, ubv, ibv, outv, sem):
        wid = lax.axis_index("s") * _NC + lax.axis_index("c")
        base = wid * bpw
        for j in range(nch):
            off = base + j * _CHUNK
            pltpu.sync_copy(uid.at[pl.ds(off, _CHUNK)], uidv.at[j])
            pltpu.sync_copy(iid.at[pl.ds(off, _CHUNK)], iidv.at[j])
        copies = []
        for j in range(nch):
            s = pl.ds(j * _CHUNK, _CHUNK)
            copies.append(pltpu.async_copy(ut.at[uidv.at[j]], urow.at[s], sem))
            copies.append(pltpu.async_copy(it.at[iidv.at[j]], irow.at[s], sem))
            copies.append(pltpu.async_copy(ubt.at[uidv.at[j]], ubv.at[s], sem))
            copies.append(pltpu.async_copy(ibt.at[iidv.at[j]], ibv.at[s], sem))
        for c in copies:
            c.wait()

        lane = lax.iota(jnp.int32, _L)

        def body(g, carry):
            rows = g * _L + lane
            acc = jnp.zeros((_L,), jnp.float32)
            for d in range(D):
                dv = jnp.full((_L,), d, jnp.int32)
                acc = acc + (plsc.load_gather(urow, [rows, dv]) *
                             plsc.load_gather(irow, [rows, dv]))
            acc = acc + plsc.load_gather(ubv, [rows])
            acc = acc + plsc.load_gather(ibv, [rows])
            outv[pl.ds(g * _L, _L)] = acc
            return carry

        lax.fori_loop(0, groups, body, 0)
        pltpu.sync_copy(outv, out.at[pl.ds(base, bpw)])

    return sc_kernel


def kernel(user_table, item_table, user_bias_table, item_bias_table,
           user_ids, item_ids):
    B = user_ids.shape[0]
    D = user_table.shape[1]
    f = _build(B, D)
    out = f(user_table, item_table,
            user_bias_table.reshape(-1), item_bias_table.reshape(-1),
            user_ids.reshape(B), item_ids.reshape(B))
    return out.reshape(B, 1)


# speedup vs baseline: 1.0023x; 1.0023x over previous
import functools

import jax
import jax.numpy as jnp
from jax import lax
from jax.experimental import pallas as pl
from jax.experimental.pallas import tpu as pltpu
from jax.experimental.pallas import tpu_sc as plsc

_NC = 2
_NS = 16
_L = 16
_CHUNK = 128


@functools.cache
def _build(B, D):
    NW = _NC * _NS
    bpw = B // NW
    nch = bpw // _CHUNK
    groups = bpw // _L
    mesh = plsc.VectorSubcoreMesh(core_axis_name="c", subcore_axis_name="s",
                                  num_cores=_NC, num_subcores=_NS)

    @functools.partial(
        pl.kernel,
        out_type=jax.ShapeDtypeStruct((B,), jnp.float32),
        mesh=mesh,
        scratch_types=[
            pltpu.VMEM((nch, _CHUNK), jnp.int32),
            pltpu.VMEM((nch, _CHUNK), jnp.int32),
            pltpu.VMEM((nch, _CHUNK), jnp.int32),
            pltpu.VMEM((nch, _CHUNK), jnp.int32),
            pltpu.VMEM((bpw, D), jnp.float32),
            pltpu.VMEM((bpw, D), jnp.float32),
            pltpu.VMEM((bpw, 32), jnp.float32),
            pltpu.VMEM((bpw, 32), jnp.float32),
            pltpu.VMEM((bpw,), jnp.float32),
            pltpu.SemaphoreType.DMA,
        ],
        compiler_params=pltpu.CompilerParams(
            needs_layout_passes=False, use_tc_tiling_on_sc=False),
    )
    def sc_kernel(ut, it, ubt, ibt, uid, iid, out,
                  uidv, iidv, usr, isr, urow, irow, ubrow, ibrow, outv, sem):
        wid = lax.axis_index("s") * _NC + lax.axis_index("c")
        base = wid * bpw
        for j in range(nch):
            off = base + j * _CHUNK
            pltpu.sync_copy(uid.at[pl.ds(off, _CHUNK)], uidv.at[j])
            pltpu.sync_copy(iid.at[pl.ds(off, _CHUNK)], iidv.at[j])
        for j in range(nch):
            for kk in range(_CHUNK // _L):
                s = pl.ds(kk * _L, _L)
                usr.at[j][s] = uidv.at[j][s] >> 5
                isr.at[j][s] = iidv.at[j][s] >> 5
        copies = []
        for j in range(nch):
            s = pl.ds(j * _CHUNK, _CHUNK)
            copies.append(pltpu.async_copy(ut.at[uidv.at[j]], urow.at[s], sem))
            copies.append(pltpu.async_copy(it.at[iidv.at[j]], irow.at[s], sem))
            copies.append(pltpu.async_copy(ubt.at[usr.at[j]], ubrow.at[s], sem))
            copies.append(pltpu.async_copy(ibt.at[isr.at[j]], ibrow.at[s], sem))
        for c in copies:
            c.wait()

        lane = lax.iota(jnp.int32, _L)

        def body(g, carry):
            rows = g * _L + lane
            chnk = rows >> 7
            pos = rows & 127
            acc = jnp.zeros((_L,), jnp.float32)
            for d in range(D):
                dv = jnp.full((_L,), d, jnp.int32)
                acc = acc + (plsc.load_gather(urow, [rows, dv]) *
                             plsc.load_gather(irow, [rows, dv]))
            ucol = plsc.load_gather(uidv, [chnk, pos]) & 31
            icol = plsc.load_gather(iidv, [chnk, pos]) & 31
            acc = acc + plsc.load_gather(ubrow, [rows, ucol])
            acc = acc + plsc.load_gather(ibrow, [rows, icol])
            outv[pl.ds(g * _L, _L)] = acc
            return carry

        lax.fori_loop(0, groups, body, 0)
        pltpu.sync_copy(outv, out.at[pl.ds(base, bpw)])

    return sc_kernel


def kernel(user_table, item_table, user_bias_table, item_bias_table,
           user_ids, item_ids):
    B = user_ids.shape[0]
    D = user_table.shape[1]
    V = user_bias_table.shape[0]
    f = _build(B, D)
    out = f(user_table, item_table,
            user_bias_table.reshape(V // 32, 32),
            item_bias_table.reshape(V // 32, 32),
            user_ids.reshape(B), item_ids.reshape(B))
    return out.reshape(B, 1)
